# Initial kernel scaffold; baseline (speedup 1.0000x reference)
import jax
import jax.numpy as jnp
from jax.experimental import pallas as pl


def kernel(embeddings, edge_index, edge_weight, W1, b1, g1, be1, W2, b2, g2, be2, W3, b3, U1, ub1, ug, ube, U2, ub2, U3, ub3, gamma_p, beta_p, cscale, mixraw):
    raise NotImplementedError("write your pallas kernel here")


# speedup vs baseline: 215.4728x; 215.4728x over previous
import functools

import jax
import jax.numpy as jnp
from jax import lax
from jax.experimental import pallas as pl
from jax.experimental.pallas import tpu as pltpu
from jax.experimental.pallas import tpu_sc as plsc

_N = 10000
_NPAD = 10240
_E = 320000
_NC = 2
_NS = 16
_NW = _NC * _NS
_EPW = _E // _NW


def _leaky(x, s):
    return jnp.where(x > 0, x, s * x)


def _scoring_body(emb, W1, b1, g1, be1, W2, b2, g2, be2, W3, b3, out):
    x = emb[...]
    z = jnp.dot(x, W1[...], preferred_element_type=jnp.float32) + b1[...]
    m = jnp.sum(z, axis=0, keepdims=True) * (1.0 / _N)
    v = jnp.sum((z - m) ** 2, axis=0, keepdims=True) * (1.0 / _N)
    h = _leaky((z - m) / jnp.sqrt(v + 1e-5) * g1[...] + be1[...], 0.2)
    z = jnp.dot(h, W2[...], preferred_element_type=jnp.float32) + b2[...]
    m = jnp.sum(z, axis=0, keepdims=True) * (1.0 / _N)
    v = jnp.sum((z - m) ** 2, axis=0, keepdims=True) * (1.0 / _N)
    h = _leaky((z - m) / jnp.sqrt(v + 1e-5) * g2[...] + be2[...], 0.2)
    s = jnp.dot(h, W3[...], preferred_element_type=jnp.float32) + b3[...]
    out[0:_N, :] = s
    out[_N:_NPAD, :] = jnp.zeros((_NPAD - _N, 1), jnp.float32)


def _scoring(emb, W1, b1, g1, be1, W2, b2, g2, be2, W3, b3):
    return pl.pallas_call(
        _scoring_body,
        out_shape=jax.ShapeDtypeStruct((_NPAD, 1), jnp.float32),
    )(emb, W1, b1, g1, be1, W2, b2, g2, be2, W3, b3)


def _edge_body(with_deg, scores_hbm, src_hbm, dst_hbm, w_hbm, out_hbm,
               scores_v, src_v, dst_v, w_v, *accs):
    cid = lax.axis_index("c")
    sid = lax.axis_index("s")
    wid = sid * _NC + cid
    base = wid * _EPW

    pltpu.sync_copy(scores_hbm, scores_v)
    pltpu.sync_copy(src_hbm.at[pl.ds(base, _EPW)], src_v)
    pltpu.sync_copy(dst_hbm.at[pl.ds(base, _EPW)], dst_v)
    pltpu.sync_copy(w_hbm.at[pl.ds(base, _EPW)], w_v)

    zero16 = jnp.zeros((16,), jnp.float32)

    def zbody(i, _):
        o = i * 16
        for a in accs:
            a[pl.ds(o, 16)] = zero16
        return 0

    lax.fori_loop(0, _NPAD // 16, zbody, 0)

    ones16 = jnp.ones((16,), jnp.float32)

    def ebody(i, _):
        o = i * 16
        isrc = src_v[pl.ds(o, 16)]
        idst = dst_v[pl.ds(o, 16)]
        wv = w_v[pl.ds(o, 16)]
        ss = plsc.load_gather(scores_v, [isrc])
        sd = plsc.load_gather(scores_v, [idst])
        es = jnp.exp(ss)
        ed = jnp.exp(sd)
        plsc.addupdate_scatter(accs[0], [idst], es)
        plsc.addupdate_scatter(accs[1], [idst], es * wv * ss)
        plsc.addupdate_scatter(accs[2], [isrc], ed)
        plsc.addupdate_scatter(accs[3], [isrc], ed * wv * sd)
        if with_deg:
            plsc.addupdate_scatter(accs[4], [idst], ones16)
        return 0

    lax.fori_loop(0, _EPW // 16, ebody, 0)

    nq = len(accs)
    for q, a in enumerate(accs):
        pltpu.sync_copy(a, out_hbm.at[pl.ds((wid * nq + q) * _NPAD, _NPAD)])


@functools.lru_cache(maxsize=None)
def _make_edge(with_deg):
    nq = 5 if with_deg else 4
    scratch = [
        pltpu.VMEM((_NPAD,), jnp.float32),
        pltpu.VMEM((_EPW,), jnp.int32),
        pltpu.VMEM((_EPW,), jnp.int32),
        pltpu.VMEM((_EPW,), jnp.float32),
    ] + [pltpu.VMEM((_NPAD,), jnp.float32) for _ in range(nq)]
    return pl.kernel(
        functools.partial(_edge_body, with_deg),
        out_type=jax.ShapeDtypeStruct((_NW * nq * _NPAD,), jnp.float32),
        mesh=plsc.VectorSubcoreMesh(core_axis_name="c", subcore_axis_name="s",
                                    num_cores=_NC, num_subcores=_NS),
        scratch_types=scratch,
        compiler_params=pltpu.CompilerParams(needs_layout_passes=False),
    )


def _edge_deg(*args):
    return _make_edge(True)(*args)


def _edge_nodeg(*args):
    return _make_edge(False)(*args)


def _node_body(alpha, final, parts, s_in_ref, orig_ref,
               U1T, ub1, ug, ube, U2T, ub2, U3T, ub3, *rest):
    if final:
        scal_ref, deg_ref, out_s = rest
    else:
        out_s, out_deg = rest

    red = jnp.sum(parts[...], axis=0)
    den_in = red[0:1]
    num_in = red[1:2]
    den_out = red[2:3]
    num_out = red[3:4]
    s_in = num_in / jnp.maximum(den_in, 1e-20)
    s_out = num_out / jnp.maximum(den_out, 1e-20)
    s = s_in_ref[...]

    mask = (lax.broadcasted_iota(jnp.int32, (1, _NPAD), 1) < _N).astype(
        jnp.float32)

    u1 = (U1T[...][:, 0:1] * s + U1T[...][:, 1:2] * s_in
          + U1T[...][:, 2:3] * s_out + ub1[...])
    u1 = u1 * mask
    m = jnp.sum(u1, axis=1, keepdims=True) * (1.0 / _N)
    v = jnp.sum(u1 * u1, axis=1, keepdims=True) * (1.0 / _N) - m * m
    h = _leaky((u1 - m) / jnp.sqrt(v + 1e-5) * ug[...] + ube[...], 0.1)
    u2 = _leaky(
        jnp.dot(U2T[...], h, preferred_element_type=jnp.float32) + ub2[...],
        0.1)
    u3 = jax.nn.sigmoid(
        jnp.dot(U3T[...], u2, preferred_element_type=jnp.float32) + ub3[...])
    sn = alpha * u3 + (1.0 - alpha) * orig_ref[...]

    if not final:
        out_s[...] = sn
        out_deg[...] = red[4:5]
    else:
        scal = scal_ref[...]
        gamma_p = scal[0, 0]
        beta_p = scal[0, 1]
        cscale = scal[0, 2]
        mixraw = scal[0, 3]
        deg = deg_ref[...]
        c_v = jnp.log(deg * cscale + 1e-6)
        c_smooth = 5.0 * jnp.tanh((gamma_p * c_v + beta_p) * 0.2)
        mix = jax.nn.sigmoid(mixraw)
        out_s[...] = mix * (sn * jax.nn.sigmoid(c_smooth)) + (1.0 - mix) * sn


def _node_first(parts, s_row, orig, U1T, ub1, ug, ube, U2T, ub2, U3T, ub3):
    return pl.pallas_call(
        functools.partial(_node_body, 0.8, False),
        out_shape=(
            jax.ShapeDtypeStruct((1, _NPAD), jnp.float32),
            jax.ShapeDtypeStruct((1, _NPAD), jnp.float32),
        ),
    )(parts, s_row, orig, U1T, ub1, ug, ube, U2T, ub2, U3T, ub3)


def _node_final(parts, s_row, orig, U1T, ub1, ug, ube, U2T, ub2, U3T, ub3,
                scal, deg_row):
    return pl.pallas_call(
        functools.partial(_node_body, 0.9, True),
        out_shape=jax.ShapeDtypeStruct((1, _NPAD), jnp.float32),
    )(parts, s_row, orig, U1T, ub1, ug, ube, U2T, ub2, U3T, ub3, scal, deg_row)


def kernel(embeddings, edge_index, edge_weight, W1, b1, g1, be1, W2, b2, g2,
           be2, W3, b3, U1, ub1, ug, ube, U2, ub2, U3, ub3, gamma_p, beta_p,
           cscale, mixraw):
    src = edge_index[0]
    dst = edge_index[1]

    scores_col = _scoring(
        embeddings, W1, b1.reshape(1, -1), g1.reshape(1, -1),
        be1.reshape(1, -1), W2, b2.reshape(1, -1), g2.reshape(1, -1),
        be2.reshape(1, -1), W3, b3.reshape(1, -1))
    s_row = scores_col.reshape(1, _NPAD)
    orig = s_row

    scal = jnp.stack([gamma_p, beta_p, cscale, mixraw]).reshape(1, 4)

    def layer_args(l):
        return (U1[l].T, ub1[l].reshape(-1, 1), ug[l].reshape(-1, 1),
                ube[l].reshape(-1, 1), U2[l].T, ub2[l].reshape(-1, 1),
                U3[l].T, ub3[l].reshape(1, 1))

    parts0 = _edge_deg(s_row.reshape(_NPAD), src, dst, edge_weight)
    parts0 = parts0.reshape(_NW, 5, _NPAD)
    s_row, deg_row = _node_first(parts0, s_row, orig, *layer_args(0))

    parts1 = _edge_nodeg(s_row.reshape(_NPAD), src, dst, edge_weight)
    parts1 = parts1.reshape(_NW, 4, _NPAD)
    fin = _node_final(parts1, s_row, orig, *layer_args(1), scal, deg_row)

    return fin.reshape(_NPAD)[:_N]


# speedup vs baseline: 267.5069x; 1.2415x over previous
import functools

import jax
import jax.numpy as jnp
from jax import lax
from jax.experimental import pallas as pl
from jax.experimental.pallas import tpu as pltpu
from jax.experimental.pallas import tpu_sc as plsc

_N = 10000
_NPAD = 10240
_E = 320000
_NC = 2
_NS = 16
_NW = _NC * _NS
_EPW = _E // _NW
_UNROLL = 5


def _leaky(x, s):
    return jnp.where(x > 0, x, s * x)


def _bn_leaky(z, g, b, slope):
    ones = jnp.ones((1, _N), jnp.float32)
    s1 = jnp.dot(ones, z, preferred_element_type=jnp.float32)
    s2 = jnp.dot(ones, z * z, preferred_element_type=jnp.float32)
    m = s1 * (1.0 / _N)
    v = s2 * (1.0 / _N) - m * m
    return _leaky((z - m) / jnp.sqrt(v + 1e-5) * g + b, slope)


def _scoring_body(emb, W1, b1, g1, be1, W2, b2, g2, be2, W3, b3, out):
    x = emb[...]
    z = jnp.dot(x, W1[...], preferred_element_type=jnp.float32) + b1[...]
    h = _bn_leaky(z, g1[...], be1[...], 0.2)
    z = jnp.dot(h, W2[...], preferred_element_type=jnp.float32) + b2[...]
    h = _bn_leaky(z, g2[...], be2[...], 0.2)
    s = jnp.dot(h, W3[...], preferred_element_type=jnp.float32) + b3[...]
    srow = s.T
    out[:, 0:_N] = srow
    out[:, _N:_NPAD] = jnp.zeros((1, _NPAD - _N), jnp.float32)


def _scoring(emb, W1, b1, g1, be1, W2, b2, g2, be2, W3, b3):
    return pl.pallas_call(
        _scoring_body,
        out_shape=jax.ShapeDtypeStruct((1, _NPAD), jnp.float32),
    )(emb, W1, b1.reshape(1, -1), g1.reshape(1, -1), be1.reshape(1, -1),
      W2, b2.reshape(1, -1), g2.reshape(1, -1), be2.reshape(1, -1),
      W3, b3.reshape(1, -1))


def _edge_body(with_deg, scores_hbm, src_hbm, dst_hbm, w_hbm, out_hbm,
               scores_v, src_v, dst_v, w_v, *accs):
    cid = lax.axis_index("c")
    sid = lax.axis_index("s")
    wid = sid * _NC + cid
    base = wid * _EPW

    pltpu.sync_copy(scores_hbm, scores_v)
    pltpu.sync_copy(src_hbm.at[pl.ds(base, _EPW)], src_v)
    pltpu.sync_copy(dst_hbm.at[pl.ds(base, _EPW)], dst_v)
    pltpu.sync_copy(w_hbm.at[pl.ds(base, _EPW)], w_v)

    zero16 = jnp.zeros((16,), jnp.float32)

    def zbody(i, _):
        o = i * 64
        for j in range(4):
            for a in accs:
                a[pl.ds(o + j * 16, 16)] = zero16
        return 0

    lax.fori_loop(0, _NPAD // 64, zbody, 0)

    ones16 = jnp.ones((16,), jnp.float32)

    def ebody(i, _):
        for j in range(_UNROLL):
            o = i * (16 * _UNROLL) + j * 16
            isrc = src_v[pl.ds(o, 16)]
            idst = dst_v[pl.ds(o, 16)]
            wv = w_v[pl.ds(o, 16)]
            ss = plsc.load_gather(scores_v, [isrc])
            sd = plsc.load_gather(scores_v, [idst])
            es = jnp.exp(ss)
            ed = jnp.exp(sd)
            plsc.addupdate_scatter(accs[0], [idst], es)
            plsc.addupdate_scatter(accs[1], [idst], es * wv * ss)
            plsc.addupdate_scatter(accs[2], [isrc], ed)
            plsc.addupdate_scatter(accs[3], [isrc], ed * wv * sd)
            if with_deg:
                plsc.addupdate_scatter(accs[4], [idst], ones16)
        return 0

    lax.fori_loop(0, _EPW // (16 * _UNROLL), ebody, 0)

    nq = len(accs)
    for q, a in enumerate(accs):
        pltpu.sync_copy(a, out_hbm.at[pl.ds((wid * nq + q) * _NPAD, _NPAD)])


@functools.lru_cache(maxsize=None)
def _make_edge(with_deg):
    nq = 5 if with_deg else 4
    scratch = [
        pltpu.VMEM((_NPAD,), jnp.float32),
        pltpu.VMEM((_EPW,), jnp.int32),
        pltpu.VMEM((_EPW,), jnp.int32),
        pltpu.VMEM((_EPW,), jnp.float32),
    ] + [pltpu.VMEM((_NPAD,), jnp.float32) for _ in range(nq)]
    return pl.kernel(
        functools.partial(_edge_body, with_deg),
        out_type=jax.ShapeDtypeStruct((_NW * nq * _NPAD,), jnp.float32),
        mesh=plsc.VectorSubcoreMesh(core_axis_name="c", subcore_axis_name="s",
                                    num_cores=_NC, num_subcores=_NS),
        scratch_types=scratch,
        compiler_params=pltpu.CompilerParams(needs_layout_passes=False),
    )


def _edge_deg(*args):
    return _make_edge(True)(*args)


def _edge_nodeg(*args):
    return _make_edge(False)(*args)


def _reduce_plane(parts, nq, q):
    o = q * _NPAD
    acc = parts[o:o + _NPAD]
    for w in range(1, _NW):
        o = (w * nq + q) * _NPAD
        acc = acc + parts[o:o + _NPAD]
    return acc.reshape(1, _NPAD)


def _node_body(alpha, final, parts_ref, s_in_ref, orig_ref,
               U1_ref, ub1_ref, ug_ref, ube_ref, U2_ref, ub2_ref, U3_ref,
               ub3_ref, *rest):
    nq = 4 if final else 5
    if final:
        scal_ref, deg_ref, out_s = rest
    else:
        out_s, out_deg = rest

    parts = parts_ref[...]
    den_in = _reduce_plane(parts, nq, 0)
    num_in = _reduce_plane(parts, nq, 1)
    den_out = _reduce_plane(parts, nq, 2)
    num_out = _reduce_plane(parts, nq, 3)
    s_in = num_in / jnp.maximum(den_in, 1e-20)
    s_out = num_out / jnp.maximum(den_out, 1e-20)
    s = s_in_ref[...]

    mask = (lax.broadcasted_iota(jnp.int32, (1, _NPAD), 1) < _N).astype(
        jnp.float32)

    U1T = U1_ref[...].T
    ub1 = ub1_ref[...].T
    ug = ug_ref[...].T
    ube = ube_ref[...].T
    U2T = U2_ref[...].T
    ub2 = ub2_ref[...].T
    U3T = U3_ref[...].T
    ub3 = ub3_ref[...]

    u1 = (U1T[:, 0:1] * s + U1T[:, 1:2] * s_in
          + U1T[:, 2:3] * s_out + ub1)
    u1 = u1 * mask
    m = jnp.sum(u1, axis=1, keepdims=True) * (1.0 / _N)
    v = jnp.sum(u1 * u1, axis=1, keepdims=True) * (1.0 / _N) - m * m
    h = _leaky((u1 - m) / jnp.sqrt(v + 1e-5) * ug + ube, 0.1)
    u2 = _leaky(
        jnp.dot(U2T, h, preferred_element_type=jnp.float32) + ub2, 0.1)
    u3 = jax.nn.sigmoid(
        jnp.dot(U3T, u2, preferred_element_type=jnp.float32) + ub3)
    sn = alpha * u3 + (1.0 - alpha) * orig_ref[...]

    if not final:
        out_s[...] = sn
        out_deg[...] = _reduce_plane(parts, nq, 4)
    else:
        scal = scal_ref[...]
        gamma_p = scal[0, 0]
        beta_p = scal[0, 1]
        cscale = scal[0, 2]
        mixraw = scal[0, 3]
        deg = deg_ref[...]
        c_v = jnp.log(deg * cscale + 1e-6)
        c_smooth = 5.0 * jnp.tanh((gamma_p * c_v + beta_p) * 0.2)
        mix = jax.nn.sigmoid(mixraw)
        out_s[...] = mix * (sn * jax.nn.sigmoid(c_smooth)) + (1.0 - mix) * sn


def _node_first(parts, s_row, orig, U1l, ub1l, ugl, ubel, U2l, ub2l, U3l,
                ub3l):
    return pl.pallas_call(
        functools.partial(_node_body, 0.8, False),
        out_shape=(
            jax.ShapeDtypeStruct((1, _NPAD), jnp.float32),
            jax.ShapeDtypeStruct((1, _NPAD), jnp.float32),
        ),
    )(parts, s_row, orig, U1l, ub1l, ugl, ubel, U2l, ub2l, U3l, ub3l)


def _node_final(parts, s_row, orig, U1l, ub1l, ugl, ubel, U2l, ub2l, U3l,
                ub3l, scal, deg_row):
    return pl.pallas_call(
        functools.partial(_node_body, 0.9, True),
        out_shape=jax.ShapeDtypeStruct((1, _NPAD), jnp.float32),
    )(parts, s_row, orig, U1l, ub1l, ugl, ubel, U2l, ub2l, U3l, ub3l,
      scal, deg_row)


def kernel(embeddings, edge_index, edge_weight, W1, b1, g1, be1, W2, b2, g2,
           be2, W3, b3, U1, ub1, ug, ube, U2, ub2, U3, ub3, gamma_p, beta_p,
           cscale, mixraw):
    src = edge_index[0]
    dst = edge_index[1]

    s_row = _scoring(embeddings, W1, b1, g1, be1, W2, b2, g2, be2, W3, b3)
    orig = s_row

    scal = jnp.stack([gamma_p, beta_p, cscale, mixraw]).reshape(1, 4)

    def layer_args(l):
        return (U1[l], ub1[l].reshape(1, -1), ug[l].reshape(1, -1),
                ube[l].reshape(1, -1), U2[l], ub2[l].reshape(1, -1),
                U3[l], ub3[l].reshape(1, 1))

    parts0 = _edge_deg(s_row.reshape(_NPAD), src, dst, edge_weight)
    s_row, deg_row = _node_first(parts0, s_row, orig, *layer_args(0))

    parts1 = _edge_nodeg(s_row.reshape(_NPAD), src, dst, edge_weight)
    fin = _node_final(parts1, s_row, orig, *layer_args(1), scal, deg_row)

    return fin.reshape(_NPAD)[:_N]
